# baseline (speedup 1.0000x reference)
import functools

import jax
import jax.numpy as jnp
from jax import lax
from jax.experimental import pallas as pl
from jax.experimental.pallas import tpu as pltpu
from jax.experimental.pallas import tpu_sc as plsc

NCODEBOOKS = 16
NCENTROIDS = 1024
SUBVECT = 256
BATCH = 4096

TB = 256


def _tc_body(x_ref, c_ref, idx_ref):
    cols = []
    for c in range(NCODEBOOKS):
        a = x_ref[:, c, :]
        w = c_ref[c, :, :]
        resp = lax.dot_general(
            a, w, (((1,), (1,)), ((), ())),
            preferred_element_type=jnp.float32)
        idx = jnp.argmax(resp, axis=1)[:, None].astype(jnp.int32)
        cols.append(idx + c * NCENTROIDS)
    idx_ref[...] = jnp.concatenate(cols, axis=1)


def _tc_codes(inputs, centroids, batch, batch_offset):
    grid = (batch // TB,)
    off = batch_offset // TB
    return pl.pallas_call(
        _tc_body,
        grid=grid,
        in_specs=[
            pl.BlockSpec((TB, NCODEBOOKS, SUBVECT), lambda i: (off + i, 0, 0)),
            pl.BlockSpec((NCODEBOOKS, NCENTROIDS, SUBVECT), lambda i: (0, 0, 0)),
        ],
        out_specs=pl.BlockSpec((TB, NCODEBOOKS), lambda i: (i, 0)),
        out_shape=jax.ShapeDtypeStruct((batch, NCODEBOOKS), jnp.int32),
    )(inputs, centroids)


def _make_sc_gather(total, row_offset):
    info = plsc.get_sparse_core_info()
    nw = info.num_cores * info.num_subcores
    b_per_w = total // nw
    chunk = min(128, b_per_w)
    nchunks = b_per_w // chunk

    mesh = plsc.VectorSubcoreMesh(core_axis_name="c", subcore_axis_name="s")

    @functools.partial(
        pl.kernel,
        mesh=mesh,
        scratch_types=[
            pltpu.VMEM((b_per_w,), jnp.int32),
            pltpu.VMEM((2, chunk, SUBVECT), jnp.float32),
            pltpu.SemaphoreType.DMA,
            pltpu.SemaphoreType.DMA,
        ],
    )
    def gather(table_hbm, idx_hbm, out_ref, idx_v, rows_v, sem0, sem1):
        wid = lax.axis_index("s") * info.num_cores + lax.axis_index("c")
        base = wid * b_per_w
        pltpu.sync_copy(idx_hbm.at[pl.ds(base, b_per_w)], idx_v)
        sems = (sem0, sem1)
        copies = [None, None]
        for j in range(nchunks):
            s = j % 2
            copies[s] = pltpu.async_copy(
                table_hbm.at[idx_v.at[pl.ds(j * chunk, chunk)]],
                rows_v.at[s], sems[s])
            if j > 0:
                p = (j - 1) % 2
                copies[p].wait()
                pltpu.sync_copy(
                    rows_v.at[p],
                    out_ref.at[pl.ds(row_offset + base + (j - 1) * chunk,
                                     chunk)])
        p = (nchunks - 1) % 2
        copies[p].wait()
        pltpu.sync_copy(
            rows_v.at[p],
            out_ref.at[pl.ds(row_offset + base + (nchunks - 1) * chunk,
                             chunk)])

    return gather


_OUT_REF = jax.new_ref(jnp.zeros((BATCH * NCODEBOOKS, SUBVECT), jnp.float32))

_SPLITS = (2304, 1792)


def kernel(inputs, centroids):
    flat_centroids = centroids.reshape((-1, SUBVECT))
    idxs = []
    row0 = 0
    for part in _SPLITS:
        idx_h = _tc_codes(inputs, centroids, part, row0)
        idxs.append(idx_h)
        gather = _make_sc_gather(part * NCODEBOOKS, row0 * NCODEBOOKS)
        gather(flat_centroids, idx_h.reshape(-1), _OUT_REF)
        row0 += part
    neighbour_idxs = jnp.concatenate(idxs, axis=0)
    out = _OUT_REF[...]
    return (neighbour_idxs, out.reshape((BATCH, NCODEBOOKS, SUBVECT)))


# speedup vs baseline: 1.2385x; 1.2385x over previous
import functools

import jax
import jax.numpy as jnp
from jax import lax
from jax.experimental import pallas as pl
from jax.experimental.pallas import tpu as pltpu
from jax.experimental.pallas import tpu_sc as plsc

NCODEBOOKS = 16
NCENTROIDS = 1024
SUBVECT = 256
BATCH = 4096

TB = 512


def _tc_body(x_ref, c_ref, idx_ref):
    cols = []
    for c in range(NCODEBOOKS):
        a = x_ref[:, c, :]
        w = c_ref[c, :, :]
        resp = lax.dot_general(
            a, w, (((1,), (1,)), ((), ())),
            preferred_element_type=jnp.float32)
        idx = jnp.argmax(resp, axis=1)[:, None].astype(jnp.int32)
        cols.append(idx + c * NCENTROIDS)
    idx_ref[...] = jnp.concatenate(cols, axis=1)


def _tc_codes(inputs, centroids, batch, batch_offset):
    grid = (batch // TB,)
    off = batch_offset // TB
    return pl.pallas_call(
        _tc_body,
        grid=grid,
        in_specs=[
            pl.BlockSpec((TB, NCODEBOOKS, SUBVECT), lambda i: (off + i, 0, 0)),
            pl.BlockSpec((NCODEBOOKS, NCENTROIDS, SUBVECT), lambda i: (0, 0, 0)),
        ],
        out_specs=pl.BlockSpec((TB, NCODEBOOKS), lambda i: (i, 0)),
        out_shape=jax.ShapeDtypeStruct((batch, NCODEBOOKS), jnp.int32),
    )(inputs, centroids)


def _make_sc_gather(total, row_offset):
    info = plsc.get_sparse_core_info()
    nw = info.num_cores * info.num_subcores
    b_per_w = total // nw
    chunk = min(128, b_per_w)
    nchunks = b_per_w // chunk

    mesh = plsc.VectorSubcoreMesh(core_axis_name="c", subcore_axis_name="s")

    @functools.partial(
        pl.kernel,
        out_type=jax.ShapeDtypeStruct((total, SUBVECT), jnp.float32),
        mesh=mesh,
        scratch_types=[
            pltpu.VMEM((b_per_w,), jnp.int32),
            pltpu.VMEM((2, chunk, SUBVECT), jnp.float32),
            pltpu.SemaphoreType.DMA,
            pltpu.SemaphoreType.DMA,
        ],
    )
    def gather(table_hbm, idx_hbm, out_ref, idx_v, rows_v, sem0, sem1):
        wid = lax.axis_index("s") * info.num_cores + lax.axis_index("c")
        base = wid * b_per_w
        pltpu.sync_copy(idx_hbm.at[pl.ds(base, b_per_w)], idx_v)
        sems = (sem0, sem1)
        copies = [None, None]
        for j in range(nchunks):
            s = j % 2
            copies[s] = pltpu.async_copy(
                table_hbm.at[idx_v.at[pl.ds(j * chunk, chunk)]],
                rows_v.at[s], sems[s])
            if j > 0:
                p = (j - 1) % 2
                copies[p].wait()
                pltpu.sync_copy(
                    rows_v.at[p],
                    out_ref.at[pl.ds(row_offset + base + (j - 1) * chunk,
                                     chunk)])
        p = (nchunks - 1) % 2
        copies[p].wait()
        pltpu.sync_copy(
            rows_v.at[p],
            out_ref.at[pl.ds(row_offset + base + (nchunks - 1) * chunk,
                             chunk)])

    return gather


def kernel(inputs, centroids):
    flat_centroids = centroids.reshape((-1, SUBVECT))
    neighbour_idxs = _tc_codes(inputs, centroids, BATCH, 0)
    gather = _make_sc_gather(BATCH * NCODEBOOKS, 0)
    out = gather(flat_centroids, neighbour_idxs.reshape(-1))
    return (neighbour_idxs, out.reshape((BATCH, NCODEBOOKS, SUBVECT)))


# speedup vs baseline: 1.2626x; 1.0194x over previous
import functools

import jax
import jax.numpy as jnp
from jax import lax
from jax.experimental import pallas as pl
from jax.experimental.pallas import tpu as pltpu
from jax.experimental.pallas import tpu_sc as plsc

NCODEBOOKS = 16
NCENTROIDS = 1024
SUBVECT = 256
BATCH = 4096

TB = 512


def _tc_body(x_ref, c_ref, idx_ref):
    rows = []
    for c in range(NCODEBOOKS):
        a = x_ref[:, c, :]
        w = c_ref[c, :, :]
        resp = lax.dot_general(
            w, a, (((1,), (1,)), ((), ())),
            preferred_element_type=jnp.float32)
        idx = jnp.argmax(resp, axis=0)[None, :].astype(jnp.int32)
        rows.append(idx + c * NCENTROIDS)
    idx_ref[...] = jnp.concatenate(rows, axis=0)


def _tc_codes(inputs, centroids, batch, batch_offset):
    grid = (batch // TB,)
    off = batch_offset // TB
    return pl.pallas_call(
        _tc_body,
        grid=grid,
        in_specs=[
            pl.BlockSpec((TB, NCODEBOOKS, SUBVECT), lambda i: (off + i, 0, 0)),
            pl.BlockSpec((NCODEBOOKS, NCENTROIDS, SUBVECT), lambda i: (0, 0, 0)),
        ],
        out_specs=pl.BlockSpec((NCODEBOOKS, TB), lambda i: (0, i)),
        out_shape=jax.ShapeDtypeStruct((NCODEBOOKS, batch), jnp.int32),
    )(inputs, centroids)


def _make_sc_gather(total, row_offset):
    info = plsc.get_sparse_core_info()
    nw = info.num_cores * info.num_subcores
    b_per_w = total // nw
    chunk = min(128, b_per_w)
    nchunks = b_per_w // chunk

    mesh = plsc.VectorSubcoreMesh(core_axis_name="c", subcore_axis_name="s")

    @functools.partial(
        pl.kernel,
        out_type=jax.ShapeDtypeStruct((total, SUBVECT), jnp.float32),
        mesh=mesh,
        scratch_types=[
            pltpu.VMEM((b_per_w,), jnp.int32),
            pltpu.VMEM((2, chunk, SUBVECT), jnp.float32),
            pltpu.SemaphoreType.DMA,
            pltpu.SemaphoreType.DMA,
        ],
    )
    def gather(table_hbm, idx_hbm, out_ref, idx_v, rows_v, sem0, sem1):
        wid = lax.axis_index("s") * info.num_cores + lax.axis_index("c")
        base = wid * b_per_w
        pltpu.sync_copy(idx_hbm.at[pl.ds(base, b_per_w)], idx_v)
        sems = (sem0, sem1)
        copies = [None, None]
        for j in range(nchunks):
            s = j % 2
            copies[s] = pltpu.async_copy(
                table_hbm.at[idx_v.at[pl.ds(j * chunk, chunk)]],
                rows_v.at[s], sems[s])
            if j > 0:
                p = (j - 1) % 2
                copies[p].wait()
                pltpu.sync_copy(
                    rows_v.at[p],
                    out_ref.at[pl.ds(row_offset + base + (j - 1) * chunk,
                                     chunk)])
        p = (nchunks - 1) % 2
        copies[p].wait()
        pltpu.sync_copy(
            rows_v.at[p],
            out_ref.at[pl.ds(row_offset + base + (nchunks - 1) * chunk,
                             chunk)])

    return gather


def kernel(inputs, centroids):
    flat_centroids = centroids.reshape((-1, SUBVECT))
    neighbour_idxs = _tc_codes(inputs, centroids, BATCH, 0).T
    gather = _make_sc_gather(BATCH * NCODEBOOKS, 0)
    out = gather(flat_centroids, neighbour_idxs.reshape(-1))
    return (neighbour_idxs, out.reshape((BATCH, NCODEBOOKS, SUBVECT)))
